# speedup vs baseline: 4.4398x; 4.4398x over previous
import functools

import jax
import jax.numpy as jnp
from jax import lax
from jax.experimental import pallas as pl
from jax.experimental.pallas import tpu as pltpu
from jax.experimental.pallas import tpu_sc as plsc

DIM = 128
N_ROWS = 4 * 8192
CHUNK = 128
_info = plsc.get_sparse_core_info()
NC, NS = _info.num_cores, _info.num_subcores
NW = NC * NS
PER_W = N_ROWS // NW
N_CHUNKS = PER_W // CHUNK

_mesh = plsc.VectorSubcoreMesh(core_axis_name="c", subcore_axis_name="s")


@functools.partial(
    pl.kernel,
    mesh=_mesh,
    out_type=(
        jax.ShapeDtypeStruct((N_ROWS, DIM), jnp.float32),
        jax.ShapeDtypeStruct((N_ROWS, DIM), jnp.float32),
    ),
    scratch_types=[
        pltpu.VMEM((N_CHUNKS, CHUNK), jnp.int32),
        pltpu.VMEM((CHUNK, DIM), jnp.float32),
        pltpu.VMEM((CHUNK, DIM), jnp.float32),
        pltpu.SemaphoreType.DMA,
    ],
)
def _gather_kernel(cos_hbm, sin_hbm, idx_hbm, out_cos, out_sin,
                   idx_v, cos_buf, sin_buf, sem):
    wid = lax.axis_index("s") * NC + lax.axis_index("c")
    base = wid * PER_W
    pltpu.sync_copy(idx_hbm.at[wid], idx_v)
    for j in range(N_CHUNKS):
        idx_row = idx_v.at[j]
        c1 = pltpu.async_copy(cos_hbm.at[idx_row], cos_buf, sem)
        c2 = pltpu.async_copy(sin_hbm.at[idx_row], sin_buf, sem)
        c1.wait()
        c2.wait()
        pltpu.sync_copy(cos_buf, out_cos.at[pl.ds(base + j * CHUNK, CHUNK)])
        pltpu.sync_copy(sin_buf, out_sin.at[pl.ds(base + j * CHUNK, CHUNK)])


def kernel(x, position_ids, cos_cached, sin_cached):
    idx = position_ids.reshape(NW, N_CHUNKS, CHUNK).astype(jnp.int32)
    out_cos, out_sin = _gather_kernel(cos_cached, sin_cached, idx)
    shape = (*position_ids.shape, DIM)
    return (out_cos.reshape(shape).astype(x.dtype),
            out_sin.reshape(shape).astype(x.dtype))


# speedup vs baseline: 5.0701x; 1.1420x over previous
import functools

import jax
import jax.numpy as jnp
from jax import lax
from jax.experimental import pallas as pl
from jax.experimental.pallas import tpu as pltpu
from jax.experimental.pallas import tpu_sc as plsc

DIM = 128
N_ROWS = 4 * 8192
CHUNK = 128
_info = plsc.get_sparse_core_info()
NC, NS = _info.num_cores, _info.num_subcores
NW = NC * NS
PER_W = N_ROWS // NW
N_CHUNKS = PER_W // CHUNK
N_JOBS = 2 * N_CHUNKS
DEPTH = 6
AHEAD = 3

_mesh = plsc.VectorSubcoreMesh(core_axis_name="c", subcore_axis_name="s")


@functools.partial(
    pl.kernel,
    mesh=_mesh,
    out_type=(
        jax.ShapeDtypeStruct((N_ROWS, DIM), jnp.float32),
        jax.ShapeDtypeStruct((N_ROWS, DIM), jnp.float32),
    ),
    scratch_types=(
        [pltpu.VMEM((N_CHUNKS, CHUNK), jnp.int32),
         pltpu.VMEM((DEPTH, CHUNK, DIM), jnp.float32)]
        + [pltpu.SemaphoreType.DMA] * (2 * DEPTH)
    ),
)
def _gather_kernel(cos_hbm, sin_hbm, idx_hbm, out_cos, out_sin,
                   idx_v, bufs, *sems):
    gsem = sems[:DEPTH]
    ssem = sems[DEPTH:]
    wid = lax.axis_index("s") * NC + lax.axis_index("c")
    base = wid * PER_W
    pltpu.sync_copy(idx_hbm.at[wid], idx_v)

    tables = (cos_hbm, sin_hbm)
    outs = (out_cos, out_sin)
    g_copies = [None] * DEPTH
    s_copies = [None] * DEPTH

    def issue_gather(k):
        sl = k % DEPTH
        chunk, tbl = k >> 1, k & 1
        g_copies[sl] = pltpu.async_copy(
            tables[tbl].at[idx_v.at[chunk]], bufs.at[sl], gsem[sl])

    for k in range(AHEAD):
        issue_gather(k)
    for k in range(N_JOBS):
        sl = k % DEPTH
        if k + AHEAD < N_JOBS:
            nsl = (k + AHEAD) % DEPTH
            if s_copies[nsl] is not None:
                s_copies[nsl].wait()
                s_copies[nsl] = None
            issue_gather(k + AHEAD)
        g_copies[sl].wait()
        chunk, tbl = k >> 1, k & 1
        s_copies[sl] = pltpu.async_copy(
            bufs.at[sl], outs[tbl].at[pl.ds(base + chunk * CHUNK, CHUNK)],
            ssem[sl])
    for sl in range(DEPTH):
        if s_copies[sl] is not None:
            s_copies[sl].wait()


def kernel(x, position_ids, cos_cached, sin_cached):
    idx = position_ids.reshape(NW, N_CHUNKS, CHUNK).astype(jnp.int32)
    out_cos, out_sin = _gather_kernel(cos_cached, sin_cached, idx)
    shape = (*position_ids.shape, DIM)
    return (out_cos.reshape(shape).astype(x.dtype),
            out_sin.reshape(shape).astype(x.dtype))


# speedup vs baseline: 5.0829x; 1.0025x over previous
import functools

import jax
import jax.numpy as jnp
from jax import lax
from jax.experimental import pallas as pl
from jax.experimental.pallas import tpu as pltpu
from jax.experimental.pallas import tpu_sc as plsc

DIM = 128
N_ROWS = 4 * 8192
CHUNK = 128
_info = plsc.get_sparse_core_info()
NC, NS = _info.num_cores, _info.num_subcores
NW = NC * NS
PER_W = N_ROWS // NW
N_CHUNKS = PER_W // CHUNK
N_JOBS = 2 * N_CHUNKS
DEPTH = 7
AHEAD = 4

_mesh = plsc.VectorSubcoreMesh(core_axis_name="c", subcore_axis_name="s")


@functools.partial(
    pl.kernel,
    mesh=_mesh,
    out_type=(
        jax.ShapeDtypeStruct((N_ROWS, DIM), jnp.float32),
        jax.ShapeDtypeStruct((N_ROWS, DIM), jnp.float32),
    ),
    scratch_types=(
        [pltpu.VMEM((N_CHUNKS, CHUNK), jnp.int32),
         pltpu.VMEM((DEPTH, CHUNK, DIM), jnp.float32)]
        + [pltpu.SemaphoreType.DMA] * (2 * DEPTH)
    ),
)
def _gather_kernel(cos_hbm, sin_hbm, idx_hbm, out_cos, out_sin,
                   idx_v, bufs, *sems):
    gsem = sems[:DEPTH]
    ssem = sems[DEPTH:]
    wid = lax.axis_index("s") * NC + lax.axis_index("c")
    base = wid * PER_W
    pltpu.sync_copy(idx_hbm.at[wid], idx_v)

    tables = (cos_hbm, sin_hbm)
    outs = (out_cos, out_sin)
    g_copies = [None] * DEPTH
    s_copies = [None] * DEPTH

    def issue_gather(k):
        sl = k % DEPTH
        chunk, tbl = k >> 1, k & 1
        g_copies[sl] = pltpu.async_copy(
            tables[tbl].at[idx_v.at[chunk]], bufs.at[sl], gsem[sl])

    for k in range(AHEAD):
        issue_gather(k)
    for k in range(N_JOBS):
        sl = k % DEPTH
        if k + AHEAD < N_JOBS:
            nsl = (k + AHEAD) % DEPTH
            if s_copies[nsl] is not None:
                s_copies[nsl].wait()
                s_copies[nsl] = None
            issue_gather(k + AHEAD)
        g_copies[sl].wait()
        chunk, tbl = k >> 1, k & 1
        s_copies[sl] = pltpu.async_copy(
            bufs.at[sl], outs[tbl].at[pl.ds(base + chunk * CHUNK, CHUNK)],
            ssem[sl])
    for sl in range(DEPTH):
        if s_copies[sl] is not None:
            s_copies[sl].wait()


def kernel(x, position_ids, cos_cached, sin_cached):
    idx = position_ids.reshape(NW, N_CHUNKS, CHUNK).astype(jnp.int32)
    out_cos, out_sin = _gather_kernel(cos_cached, sin_cached, idx)
    shape = (*position_ids.shape, DIM)
    return (out_cos.reshape(shape).astype(x.dtype),
            out_sin.reshape(shape).astype(x.dtype))
